# Initial kernel scaffold; baseline (speedup 1.0000x reference)
import jax
import jax.numpy as jnp
from jax.experimental import pallas as pl


def kernel(hidden_states, W1, b1, g1, beta1, W2, b2, g2, beta2, W3, b3, task_embeddings):
    raise NotImplementedError("write your pallas kernel here")


# speedup vs baseline: 1.0496x; 1.0496x over previous
import jax
import jax.numpy as jnp
from jax.experimental import pallas as pl

HIDDEN = 2048
MEM = 100000
EMB = 128
BATCH = 1024

TILE = 2048
NTILES = (MEM + TILE - 1) // TILE

_DN = (((1,), (1,)), ((), ()))


def _ln_gelu(y, g, beta):
    mu = jnp.mean(y, axis=1, keepdims=True)
    d = y - mu
    var = jnp.mean(d * d, axis=1, keepdims=True)
    z = d / jnp.sqrt(var + 1e-5) * g + beta
    return 0.5 * z * (1.0 + jax.lax.erf(z * (2.0 ** -0.5)))


def _encoder_body(x_ref, w1_ref, b1_ref, g1_ref, be1_ref,
                  w2_ref, b2_ref, g2_ref, be2_ref,
                  w3_ref, b3_ref, e_ref, en_ref):
    x = x_ref[...]
    y = jax.lax.dot_general(x, w1_ref[...], _DN,
                            preferred_element_type=jnp.float32) + b1_ref[...]
    y = _ln_gelu(y, g1_ref[...], be1_ref[...])
    y = jax.lax.dot_general(y, w2_ref[...], _DN,
                            preferred_element_type=jnp.float32) + b2_ref[...]
    y = _ln_gelu(y, g2_ref[...], be2_ref[...])
    e = jax.lax.dot_general(y, w3_ref[...], _DN,
                            preferred_element_type=jnp.float32) + b3_ref[...]
    n = jnp.sqrt(jnp.sum(e * e, axis=1, keepdims=True))
    e = e / jnp.maximum(n, 1e-12)
    e_ref[...] = e
    en_ref[...] = jnp.sqrt(jnp.sum(e * e, axis=1, keepdims=True))


def _retrieve_body(e_ref, en_ref, t_ref, val_ref, idx_ref):
    i = pl.program_id(0)
    t = t_ref[...]
    num = jax.lax.dot_general(t, e_ref[...], _DN,
                              preferred_element_type=jnp.float32)
    tn = jnp.sqrt(jnp.sum(t * t, axis=1, keepdims=True))
    denom = jnp.maximum(tn * en_ref[...], 1e-8)
    sims = num / denom
    rid = jax.lax.broadcasted_iota(jnp.int32, sims.shape, 0) + i * TILE
    sims = jnp.where(rid < MEM, sims, -jnp.inf)
    tmax = jnp.max(sims, axis=0, keepdims=True)
    targ = jnp.argmax(sims, axis=0).astype(jnp.int32)[None, :] + i * TILE

    @pl.when(i == 0)
    def _():
        val_ref[...] = tmax
        idx_ref[...] = targ

    @pl.when(i > 0)
    def _():
        prev = val_ref[...]
        better = tmax > prev
        val_ref[...] = jnp.where(better, tmax, prev)
        idx_ref[...] = jnp.where(better, targ, idx_ref[...])


def kernel(hidden_states, W1, b1, g1, beta1, W2, b2, g2, beta2, W3, b3,
           task_embeddings):
    x = hidden_states[:, 0]
    row = lambda v: v.reshape(1, -1)

    e, en = pl.pallas_call(
        _encoder_body,
        out_shape=(
            jax.ShapeDtypeStruct((BATCH, EMB), jnp.float32),
            jax.ShapeDtypeStruct((BATCH, 1), jnp.float32),
        ),
    )(x, W1, row(b1), row(g1), row(beta1),
      W2, row(b2), row(g2), row(beta2), W3, row(b3))

    en_t = en.reshape(1, BATCH)

    val, idx = pl.pallas_call(
        _retrieve_body,
        grid=(NTILES,),
        in_specs=[
            pl.BlockSpec((BATCH, EMB), lambda i: (0, 0)),
            pl.BlockSpec((1, BATCH), lambda i: (0, 0)),
            pl.BlockSpec((TILE, EMB), lambda i: (i, 0)),
        ],
        out_specs=(
            pl.BlockSpec((1, BATCH), lambda i: (0, 0)),
            pl.BlockSpec((1, BATCH), lambda i: (0, 0)),
        ),
        out_shape=(
            jax.ShapeDtypeStruct((1, BATCH), jnp.float32),
            jax.ShapeDtypeStruct((1, BATCH), jnp.int32),
        ),
    )(e, en_t, task_embeddings)

    return val.reshape(BATCH), idx.reshape(BATCH)


# speedup vs baseline: 1.4172x; 1.3503x over previous
import jax
import jax.numpy as jnp
from jax.experimental import pallas as pl

HIDDEN = 2048
MEM = 100000
EMB = 128
BATCH = 1024

TILE = 2000
NTILES = MEM // TILE

_DN = (((1,), (1,)), ((), ()))


def _ln_gelu(y, g, beta):
    mu = jnp.mean(y, axis=1, keepdims=True)
    d = y - mu
    var = jnp.mean(d * d, axis=1, keepdims=True)
    z = d / jnp.sqrt(var + 1e-5) * g + beta
    return 0.5 * z * (1.0 + jax.lax.erf(z * (2.0 ** -0.5)))


def _encoder_body(x_ref, w1_ref, b1_ref, g1_ref, be1_ref,
                  w2_ref, b2_ref, g2_ref, be2_ref,
                  w3_ref, b3_ref, e_ref, en_ref):
    x = x_ref[...]
    y = jax.lax.dot_general(x, w1_ref[...], _DN,
                            preferred_element_type=jnp.float32) + b1_ref[...]
    y = _ln_gelu(y, g1_ref[...], be1_ref[...])
    y = jax.lax.dot_general(y, w2_ref[...], _DN,
                            preferred_element_type=jnp.float32) + b2_ref[...]
    y = _ln_gelu(y, g2_ref[...], be2_ref[...])
    e = jax.lax.dot_general(y, w3_ref[...], _DN,
                            preferred_element_type=jnp.float32) + b3_ref[...]
    n = jnp.sqrt(jnp.sum(e * e, axis=1, keepdims=True))
    e = e / jnp.maximum(n, 1e-12)
    e_ref[...] = e
    en_ref[...] = jnp.sqrt(jnp.sum(e * e, axis=1, keepdims=True))


def _retrieve_body(e_ref, en_ref, t_ref, val_ref, idx_ref):
    i = pl.program_id(0)
    t = t_ref[...]
    num = jax.lax.dot_general(t, e_ref[...], _DN,
                              preferred_element_type=jnp.float32)
    tn = jnp.sqrt(jnp.sum(t * t, axis=1, keepdims=True))
    scaled = num * (1.0 / jnp.maximum(tn, 1e-8))
    tmax = jnp.max(scaled, axis=0, keepdims=True)
    targ = (jnp.argmax(scaled, axis=0).astype(jnp.int32)
            + i * TILE)[None, :]

    @pl.when(i == 0)
    def _():
        val_ref[...] = tmax
        idx_ref[...] = targ

    @pl.when(i > 0)
    def _():
        prev = val_ref[...]
        better = tmax > prev
        val_ref[...] = jnp.where(better, tmax, prev)
        idx_ref[...] = jnp.where(better, targ, idx_ref[...])

    @pl.when(i == NTILES - 1)
    def _():
        val_ref[...] = val_ref[...] / jnp.maximum(en_ref[...], 1e-30)


def kernel(hidden_states, W1, b1, g1, beta1, W2, b2, g2, beta2, W3, b3,
           task_embeddings):
    x = hidden_states[:, 0]
    row = lambda v: v.reshape(1, -1)

    e, en = pl.pallas_call(
        _encoder_body,
        out_shape=(
            jax.ShapeDtypeStruct((BATCH, EMB), jnp.float32),
            jax.ShapeDtypeStruct((BATCH, 1), jnp.float32),
        ),
    )(x, W1, row(b1), row(g1), row(beta1),
      W2, row(b2), row(g2), row(beta2), W3, row(b3))

    en_t = en.reshape(1, BATCH)

    val, idx = pl.pallas_call(
        _retrieve_body,
        grid=(NTILES,),
        in_specs=[
            pl.BlockSpec((BATCH, EMB), lambda i: (0, 0)),
            pl.BlockSpec((1, BATCH), lambda i: (0, 0)),
            pl.BlockSpec((TILE, EMB), lambda i: (i, 0)),
        ],
        out_specs=(
            pl.BlockSpec((1, BATCH), lambda i: (0, 0)),
            pl.BlockSpec((1, BATCH), lambda i: (0, 0)),
        ),
        out_shape=(
            jax.ShapeDtypeStruct((1, BATCH), jnp.float32),
            jax.ShapeDtypeStruct((1, BATCH), jnp.int32),
        ),
    )(e, en_t, task_embeddings)

    return val.reshape(BATCH), idx.reshape(BATCH)


# speedup vs baseline: 1.4612x; 1.0310x over previous
import jax
import jax.numpy as jnp
from jax.experimental import pallas as pl

HIDDEN = 2048
MEM = 100000
EMB = 128
BATCH = 1024

TILE = 4000
NTILES = MEM // TILE

_DN = (((1,), (1,)), ((), ()))


def _ln_gelu(y, g, beta):
    mu = jnp.mean(y, axis=1, keepdims=True)
    d = y - mu
    var = jnp.mean(d * d, axis=1, keepdims=True)
    z = d / jnp.sqrt(var + 1e-5) * g + beta
    return 0.5 * z * (1.0 + jax.lax.erf(z * (2.0 ** -0.5)))


def _encoder_body(x_ref, w1_ref, b1_ref, g1_ref, be1_ref,
                  w2_ref, b2_ref, g2_ref, be2_ref,
                  w3_ref, b3_ref, e_ref, en_ref):
    x = x_ref[...]
    y = jax.lax.dot_general(x, w1_ref[...], _DN,
                            preferred_element_type=jnp.float32) + b1_ref[...]
    y = _ln_gelu(y, g1_ref[...], be1_ref[...])
    y = jax.lax.dot_general(y, w2_ref[...], _DN,
                            preferred_element_type=jnp.float32) + b2_ref[...]
    y = _ln_gelu(y, g2_ref[...], be2_ref[...])
    e = jax.lax.dot_general(y, w3_ref[...], _DN,
                            preferred_element_type=jnp.float32) + b3_ref[...]
    n = jnp.sqrt(jnp.sum(e * e, axis=1, keepdims=True))
    e = e / jnp.maximum(n, 1e-12)
    e_ref[...] = e
    en_ref[...] = jnp.sqrt(jnp.sum(e * e, axis=1, keepdims=True))


def _retrieve_body(e_ref, en_ref, t_ref, val_ref, idx_ref):
    i = pl.program_id(0)
    t = t_ref[...]
    tn = jnp.sqrt(jnp.sum(t * t, axis=1, keepdims=True))
    num = jax.lax.dot_general(t, e_ref[...], _DN,
                              preferred_element_type=jnp.float32)
    scaled = num * (1.0 / jnp.maximum(tn, 1e-8))
    tmax = jnp.max(scaled, axis=0, keepdims=True)
    targ = (jnp.argmax(scaled, axis=0).astype(jnp.int32)
            + i * TILE)[None, :]

    @pl.when(i == 0)
    def _():
        val_ref[...] = tmax
        idx_ref[...] = targ

    @pl.when(i > 0)
    def _():
        prev = val_ref[...]
        better = tmax > prev
        val_ref[...] = jnp.where(better, tmax, prev)
        idx_ref[...] = jnp.where(better, targ, idx_ref[...])

    @pl.when(i == NTILES - 1)
    def _():
        val_ref[...] = val_ref[...] / jnp.maximum(en_ref[...], 1e-30)


def kernel(hidden_states, W1, b1, g1, beta1, W2, b2, g2, beta2, W3, b3,
           task_embeddings):
    x = hidden_states[:, 0]
    row = lambda v: v.reshape(1, -1)

    e, en = pl.pallas_call(
        _encoder_body,
        out_shape=(
            jax.ShapeDtypeStruct((BATCH, EMB), jnp.float32),
            jax.ShapeDtypeStruct((BATCH, 1), jnp.float32),
        ),
    )(x, W1, row(b1), row(g1), row(beta1),
      W2, row(b2), row(g2), row(beta2), W3, row(b3))

    en_t = en.reshape(1, BATCH)

    val, idx = pl.pallas_call(
        _retrieve_body,
        grid=(NTILES,),
        in_specs=[
            pl.BlockSpec((BATCH, EMB), lambda i: (0, 0)),
            pl.BlockSpec((1, BATCH), lambda i: (0, 0)),
            pl.BlockSpec((TILE, EMB), lambda i: (i, 0)),
        ],
        out_specs=(
            pl.BlockSpec((1, BATCH), lambda i: (0, 0)),
            pl.BlockSpec((1, BATCH), lambda i: (0, 0)),
        ),
        out_shape=(
            jax.ShapeDtypeStruct((1, BATCH), jnp.float32),
            jax.ShapeDtypeStruct((1, BATCH), jnp.int32),
        ),
    )(e, en_t, task_embeddings)

    return val.reshape(BATCH), idx.reshape(BATCH)
